# Initial kernel scaffold; baseline (speedup 1.0000x reference)
import jax
import jax.numpy as jnp
from jax.experimental import pallas as pl


def kernel(input_ids, id_map, table):
    raise NotImplementedError("write your pallas kernel here")


# speedup vs baseline: 12.3324x; 12.3324x over previous
import functools

import jax
import jax.numpy as jnp
from jax import lax
from jax.experimental import pallas as pl
from jax.experimental.pallas import tpu as pltpu
from jax.experimental.pallas import tpu_sc as plsc

_NC = 2
_NS = 16
_NW = _NC * _NS
_CHUNK = 128


def _make_row_gather(n_idx, d):
    per_w = n_idx // _NW
    n_chunks = per_w // _CHUNK
    mesh = plsc.VectorSubcoreMesh(core_axis_name="c", subcore_axis_name="s")

    @functools.partial(
        pl.kernel,
        mesh=mesh,
        out_type=jax.ShapeDtypeStruct((n_idx, d), jnp.float32),
        scratch_types=[
            pltpu.VMEM((_CHUNK,), jnp.int32),
            pltpu.VMEM((_CHUNK, d), jnp.float32),
            pltpu.SemaphoreType.DMA,
        ],
        compiler_params=pltpu.CompilerParams(use_tc_tiling_on_sc=False),
    )
    def k(src_hbm, idx_hbm, out_hbm, idx_v, rows_v, sem):
        wid = lax.axis_index("s") * _NC + lax.axis_index("c")
        base = wid * per_w

        def body(j, carry):
            off = base + j * _CHUNK
            pltpu.sync_copy(idx_hbm.at[pl.ds(off, _CHUNK)], idx_v)
            pltpu.async_copy(src_hbm.at[idx_v], rows_v, sem).wait()
            pltpu.sync_copy(rows_v, out_hbm.at[pl.ds(off, _CHUNK)])
            return carry

        lax.fori_loop(0, n_chunks, body, 0)

    return k


def kernel(input_ids, id_map, table):
    b, h = input_ids.shape
    v, d = table.shape
    ids_flat = input_ids.reshape(-1).astype(jnp.int32)
    idmap = id_map.astype(jnp.int32)

    grain = _NW * _CHUNK
    v_pad = ((v + grain - 1) // grain) * grain
    if v_pad != v:
        idmap = jnp.concatenate(
            [idmap, jnp.zeros((v_pad - v,), jnp.int32)])

    permute = _make_row_gather(v_pad, d)
    ptable = permute(table, idmap)

    gather = _make_row_gather(b * h, d)
    out = gather(ptable, ids_flat)
    return out.reshape(b, h, d)


# speedup vs baseline: 16.7908x; 1.3615x over previous
import functools

import jax
import jax.numpy as jnp
from jax import lax
from jax.experimental import pallas as pl
from jax.experimental.pallas import tpu as pltpu
from jax.experimental.pallas import tpu_sc as plsc

_NC = 2
_NS = 16
_NW = _NC * _NS
_CHUNK = 128
_K = 5
_SUP = _K * _CHUNK

_MESH = dict(core_axis_name="c", subcore_axis_name="s")
_PARAMS = dict(compiler_params=pltpu.CompilerParams(use_tc_tiling_on_sc=False))


def _make_row_gather_simple(n_idx, d):
    per_w = n_idx // _NW
    n_chunks = per_w // _CHUNK

    @functools.partial(
        pl.kernel,
        mesh=plsc.VectorSubcoreMesh(**_MESH),
        out_type=jax.ShapeDtypeStruct((n_idx, d), jnp.float32),
        scratch_types=[
            pltpu.VMEM((_CHUNK,), jnp.int32),
            pltpu.VMEM((_CHUNK, d), jnp.float32),
            pltpu.SemaphoreType.DMA,
        ],
        **_PARAMS,
    )
    def k(src_hbm, idx_hbm, out_hbm, idx_v, rows_v, sem):
        wid = lax.axis_index("s") * _NC + lax.axis_index("c")
        row_base = wid * n_chunks

        def body(j, carry):
            pltpu.sync_copy(idx_hbm.at[row_base + j], idx_v)
            pltpu.async_copy(src_hbm.at[idx_v], rows_v, sem).wait()
            pltpu.sync_copy(
                rows_v,
                out_hbm.at[pl.ds((row_base + j) * _CHUNK, _CHUNK)])
            return carry

        lax.fori_loop(0, n_chunks, body, 0)

    return k


def _make_row_gather_pipelined(n_idx, d):
    per_w = n_idx // _NW
    n_sup = per_w // _SUP

    @functools.partial(
        pl.kernel,
        mesh=plsc.VectorSubcoreMesh(**_MESH),
        out_type=jax.ShapeDtypeStruct((n_idx, d), jnp.float32),
        scratch_types=[
            pltpu.VMEM((2, _K, _CHUNK), jnp.int32),
            pltpu.VMEM((2, _SUP, d), jnp.float32),
            pltpu.SemaphoreType.DMA,
            pltpu.SemaphoreType.DMA,
            pltpu.SemaphoreType.DMA,
            pltpu.SemaphoreType.DMA,
        ],
        **_PARAMS,
    )
    def k(src_hbm, idx_hbm, out_hbm, idx_v, rows_v, g0, g1, o0, o1):
        wid = lax.axis_index("s") * _NC + lax.axis_index("c")
        base = wid * per_w
        row_base = wid * (per_w // _CHUNK)
        sem_g = (g0, g1)
        sem_o = (o0, o1)

        def load_idx(g, b):
            pltpu.sync_copy(idx_hbm.at[pl.ds(row_base + g * _K, _K)],
                            idx_v.at[b])

        def fire_gathers(b):
            for j in range(_K):
                pltpu.async_copy(
                    src_hbm.at[idx_v.at[b, j]],
                    rows_v.at[b, pl.ds(j * _CHUNK, _CHUNK)],
                    sem_g[b],
                )

        def drain_gathers(b):
            pltpu.make_async_copy(
                out_hbm.at[pl.ds(base, _SUP)], rows_v.at[b], sem_g[b]
            ).wait()

        for b in range(2):
            load_idx(b, b)
            fire_gathers(b)

        def body(i, carry):
            for b in range(2):
                g = 2 * i + b
                drain_gathers(b)
                store = pltpu.async_copy(
                    rows_v.at[b], out_hbm.at[pl.ds(base + g * _SUP, _SUP)],
                    sem_o[b])

                @pl.when(g + 2 < n_sup)
                def _():
                    load_idx(g + 2, b)

                store.wait()

                @pl.when(g + 2 < n_sup)
                def _():
                    fire_gathers(b)

            return carry

        lax.fori_loop(0, n_sup // 2, body, 0)

    return k


def kernel(input_ids, id_map, table):
    b, h = input_ids.shape
    v, d = table.shape
    ids_flat = input_ids.reshape(-1, _CHUNK).astype(jnp.int32)
    idmap = id_map.astype(jnp.int32)

    grain = _NW * _CHUNK
    v_pad = ((v + grain - 1) // grain) * grain
    if v_pad != v:
        idmap = jnp.concatenate(
            [idmap, jnp.zeros((v_pad - v,), jnp.int32)])
    idmap = idmap.reshape(-1, _CHUNK)

    permute = _make_row_gather_simple(v_pad, d)
    ptable = permute(table, idmap)

    gather = _make_row_gather_pipelined(b * h, d)
    out = gather(ptable, ids_flat)
    return out.reshape(b, h, d)


# speedup vs baseline: 16.7989x; 1.0005x over previous
import functools

import jax
import jax.numpy as jnp
from jax import lax
from jax.experimental import pallas as pl
from jax.experimental.pallas import tpu as pltpu
from jax.experimental.pallas import tpu_sc as plsc

_NC = 2
_NS = 16
_NW = _NC * _NS
_CHUNK = 128
_K = 5
_SUP = _K * _CHUNK

_MESH = dict(core_axis_name="c", subcore_axis_name="s")
_PARAMS = dict(compiler_params=pltpu.CompilerParams(use_tc_tiling_on_sc=False))


def _make_row_gather_simple(n_idx, d):
    per_w = n_idx // _NW
    n_chunks = per_w // _CHUNK

    @functools.partial(
        pl.kernel,
        mesh=plsc.VectorSubcoreMesh(**_MESH),
        out_type=jax.ShapeDtypeStruct((n_idx, d), jnp.float32),
        scratch_types=[
            pltpu.VMEM((_CHUNK,), jnp.int32),
            pltpu.VMEM((_CHUNK, d), jnp.float32),
            pltpu.SemaphoreType.DMA,
        ],
        **_PARAMS,
    )
    def k(src_hbm, idx_hbm, out_hbm, idx_v, rows_v, sem):
        wid = lax.axis_index("s") * _NC + lax.axis_index("c")
        row_base = wid * n_chunks

        def body(j, carry):
            pltpu.sync_copy(idx_hbm.at[row_base + j], idx_v)
            pltpu.async_copy(src_hbm.at[idx_v], rows_v, sem).wait()
            pltpu.sync_copy(
                rows_v,
                out_hbm.at[pl.ds((row_base + j) * _CHUNK, _CHUNK)])
            return carry

        lax.fori_loop(0, n_chunks, body, 0)

    return k


_NB = 4


def _make_row_gather_pipelined(nb, h, d):
    per_w = nb // _NW
    n_sup = per_w // _NB
    sup_tok = _NB * h
    pieces = []
    off = 0
    while off < h:
        sz = min(_CHUNK, h - off)
        assert sz % 8 == 0 and off % 8 == 0
        pieces.append((off, sz))
        off += sz

    @functools.partial(
        pl.kernel,
        mesh=plsc.VectorSubcoreMesh(**_MESH),
        out_type=jax.ShapeDtypeStruct((nb, h, d), jnp.float32),
        scratch_types=[
            pltpu.VMEM((2, sup_tok), jnp.int32),
            pltpu.VMEM((2, _NB, h, d), jnp.float32),
            pltpu.SemaphoreType.DMA,
            pltpu.SemaphoreType.DMA,
            pltpu.SemaphoreType.DMA,
            pltpu.SemaphoreType.DMA,
        ],
        **_PARAMS,
    )
    def k(src_hbm, idx_hbm, out_hbm, idx_v, rows_v, g0, g1, o0, o1):
        wid = lax.axis_index("s") * _NC + lax.axis_index("c")
        base = wid * per_w
        sem_g = (g0, g1)
        sem_o = (o0, o1)

        def load_idx(g, b):
            tok0 = (base + g * _NB) * h
            pltpu.sync_copy(idx_hbm.at[pl.ds(tok0, sup_tok)], idx_v.at[b])

        def fire_gathers(b):
            for r in range(_NB):
                for h_off, sz in pieces:
                    pltpu.async_copy(
                        src_hbm.at[idx_v.at[b, pl.ds(r * h + h_off, sz)]],
                        rows_v.at[b, r, pl.ds(h_off, sz)],
                        sem_g[b],
                    )

        def drain_gathers(b):
            pltpu.make_async_copy(
                out_hbm.at[pl.ds(base, _NB)], rows_v.at[b], sem_g[b]
            ).wait()

        for b in range(2):
            load_idx(b, b)
            fire_gathers(b)

        def body(i, carry):
            for b in range(2):
                g = 2 * i + b
                drain_gathers(b)
                store = pltpu.async_copy(
                    rows_v.at[b], out_hbm.at[pl.ds(base + g * _NB, _NB)],
                    sem_o[b])

                @pl.when(g + 2 < n_sup)
                def _():
                    load_idx(g + 2, b)

                store.wait()

                @pl.when(g + 2 < n_sup)
                def _():
                    fire_gathers(b)

            return carry

        lax.fori_loop(0, n_sup // 2, body, 0)

    return k


def kernel(input_ids, id_map, table):
    b, h = input_ids.shape
    v, d = table.shape
    ids_flat = input_ids.reshape(-1).astype(jnp.int32)
    idmap = id_map.astype(jnp.int32)

    grain = _NW * _CHUNK
    v_pad = ((v + grain - 1) // grain) * grain
    if v_pad != v:
        idmap = jnp.concatenate(
            [idmap, jnp.zeros((v_pad - v,), jnp.int32)])
    idmap = idmap.reshape(-1, _CHUNK)

    permute = _make_row_gather_simple(v_pad, d)
    ptable = permute(table, idmap)

    gather = _make_row_gather_pipelined(b, h, d)
    return gather(ptable, ids_flat)


# speedup vs baseline: 21.2310x; 1.2638x over previous
import functools

import jax
import jax.numpy as jnp
from jax import lax
from jax.experimental import pallas as pl
from jax.experimental.pallas import tpu as pltpu
from jax.experimental.pallas import tpu_sc as plsc

_NC = 2
_NS = 16
_NW = _NC * _NS
_CHUNK = 128

_MESH = dict(core_axis_name="c", subcore_axis_name="s")
_PARAMS = dict(compiler_params=pltpu.CompilerParams(use_tc_tiling_on_sc=True))


def _make_permute(n_idx, dp):
    per_w = n_idx // _NW
    n_chunks = per_w // _CHUNK

    @functools.partial(
        pl.kernel,
        mesh=plsc.VectorSubcoreMesh(**_MESH),
        out_type=jax.ShapeDtypeStruct((n_idx, dp), jnp.float32),
        scratch_types=[
            pltpu.VMEM((_CHUNK,), jnp.int32),
            pltpu.VMEM((_CHUNK, dp), jnp.float32),
            pltpu.SemaphoreType.DMA,
        ],
        **_PARAMS,
    )
    def k(src_hbm, idx_hbm, out_hbm, idx_v, rows_v, sem):
        wid = lax.axis_index("s") * _NC + lax.axis_index("c")
        base = wid * per_w

        def body(j, carry):
            off = base + j * _CHUNK
            pltpu.sync_copy(idx_hbm.at[pl.ds(off, _CHUNK)], idx_v)
            pltpu.async_copy(src_hbm.at[idx_v], rows_v, sem).wait()
            pltpu.sync_copy(rows_v, out_hbm.at[pl.ds(off, _CHUNK)])
            return carry

        lax.fori_loop(0, n_chunks, body, 0)

    return k


_SUP = 256
_IDXBLK = 2048


def _make_gather(n_tok, d, dp):
    per_w = n_tok // _NW
    n_sup = per_w // _SUP
    sup_per_blk = _IDXBLK // _SUP

    @functools.partial(
        pl.kernel,
        mesh=plsc.VectorSubcoreMesh(**_MESH),
        out_type=jax.ShapeDtypeStruct((n_tok, dp), jnp.float32),
        scratch_types=[
            pltpu.VMEM((2, _IDXBLK), jnp.int32),
            pltpu.VMEM((2, _SUP, dp), jnp.float32),
            pltpu.SemaphoreType.DMA,
            pltpu.SemaphoreType.DMA,
            pltpu.SemaphoreType.DMA,
            pltpu.SemaphoreType.DMA,
        ],
        **_PARAMS,
    )
    def k(src_hbm, idx_hbm, out_hbm, idx_v, rows_v, g0, g1, o0, o1):
        wid = lax.axis_index("s") * _NC + lax.axis_index("c")
        base = wid * per_w
        sem_g = (g0, g1)
        sem_o = (o0, o1)

        def load_idx_blk(blk, ib):
            pltpu.sync_copy(
                idx_hbm.at[pl.ds(base + blk * _IDXBLK, _IDXBLK)],
                idx_v.at[ib])

        def fire_gathers(g, b):
            ib = (g // sup_per_blk) % 2
            loc = (g % sup_per_blk) * _SUP
            for j in range(_SUP // _CHUNK):
                pltpu.async_copy(
                    src_hbm.at[idx_v.at[ib, pl.ds(loc + j * _CHUNK, _CHUNK)]],
                    rows_v.at[b, pl.ds(j * _CHUNK, _CHUNK)],
                    sem_g[b],
                )

        def drain_gathers(b):
            pltpu.make_async_copy(
                src_hbm.at[pl.ds(0, _SUP)], rows_v.at[b], sem_g[b]
            ).wait()

        load_idx_blk(0, 0)
        for b in range(2):
            fire_gathers(b, b)

        def body(i, carry):
            for b in range(2):
                g = 2 * i + b
                drain_gathers(b)
                store = pltpu.async_copy(
                    rows_v.at[b],
                    out_hbm.at[pl.ds(base + g * _SUP, _SUP)],
                    sem_o[b])

                @pl.when(
                    jnp.logical_and(
                        (g + 2) % sup_per_blk == 0,
                        (g + 2) // sup_per_blk < n_sup // sup_per_blk))
                def _():
                    load_idx_blk((g + 2) // sup_per_blk,
                                 ((g + 2) // sup_per_blk) % 2)

                store.wait()

                @pl.when(g + 2 < n_sup)
                def _():
                    fire_gathers(g + 2, b)

            return carry

        lax.fori_loop(0, n_sup // 2, body, 0)

    return k


def kernel(input_ids, id_map, table):
    b, h = input_ids.shape
    v, d = table.shape
    ids_flat = input_ids.reshape(-1).astype(jnp.int32)
    idmap = id_map.astype(jnp.int32)

    grain = _NW * _CHUNK
    v_pad = ((v + grain - 1) // grain) * grain
    if v_pad != v:
        idmap = jnp.concatenate(
            [idmap, jnp.zeros((v_pad - v,), jnp.int32)])

    table128 = jnp.pad(table, ((0, 0), (0, _CHUNK - d)))

    permute = _make_permute(v_pad, _CHUNK)
    ptable = permute(table128, idmap)

    gather = _make_gather(b * h, d, _CHUNK)
    out = gather(ptable, ids_flat)
    return out[:, :d].reshape(b, h, d)


# speedup vs baseline: 30.9476x; 1.4577x over previous
import functools

import jax
import jax.numpy as jnp
from jax import lax
from jax.experimental import pallas as pl
from jax.experimental.pallas import tpu as pltpu
from jax.experimental.pallas import tpu_sc as plsc

_NC = 2
_NS = 16
_NW = _NC * _NS
_CHUNK = 128

_MESH = dict(core_axis_name="c", subcore_axis_name="s")
_PARAMS = dict(compiler_params=pltpu.CompilerParams(use_tc_tiling_on_sc=False))


def _make_permute(n_idx, dp):
    per_w = n_idx // _NW
    n_chunks = per_w // _CHUNK

    @functools.partial(
        pl.kernel,
        mesh=plsc.VectorSubcoreMesh(**_MESH),
        out_type=jax.ShapeDtypeStruct((n_idx, dp), jnp.float32),
        scratch_types=[
            pltpu.VMEM((_CHUNK,), jnp.int32),
            pltpu.VMEM((_CHUNK, dp), jnp.float32),
            pltpu.SemaphoreType.DMA,
        ],
        **_PARAMS,
    )
    def k(src_hbm, idx_hbm, out_hbm, idx_v, rows_v, sem):
        wid = lax.axis_index("s") * _NC + lax.axis_index("c")
        base = wid * per_w

        def body(j, carry):
            off = base + j * _CHUNK
            pltpu.sync_copy(idx_hbm.at[pl.ds(off, _CHUNK)], idx_v)
            pltpu.async_copy(src_hbm.at[idx_v], rows_v, sem).wait()
            pltpu.sync_copy(rows_v, out_hbm.at[pl.ds(off, _CHUNK)])
            return carry

        lax.fori_loop(0, n_chunks, body, 0)

    return k


_SUP = 256
_IDXBLK = 2048


def _make_gather(n_tok, d, dp):
    per_w = n_tok // _NW
    n_sup = per_w // _SUP
    sup_per_blk = _IDXBLK // _SUP

    @functools.partial(
        pl.kernel,
        mesh=plsc.VectorSubcoreMesh(**_MESH),
        out_type=jax.ShapeDtypeStruct((n_tok, dp), jnp.float32),
        scratch_types=[
            pltpu.VMEM((2, _IDXBLK), jnp.int32),
            pltpu.VMEM((2, _SUP, d), jnp.float32),
            pltpu.SemaphoreType.DMA,
            pltpu.SemaphoreType.DMA,
            pltpu.SemaphoreType.DMA,
            pltpu.SemaphoreType.DMA,
        ],
        **_PARAMS,
    )
    def k(src_hbm, idx_hbm, out_hbm, idx_v, rows_v, g0, g1, o0, o1):
        wid = lax.axis_index("s") * _NC + lax.axis_index("c")
        base = wid * per_w
        sem_g = (g0, g1)
        sem_o = (o0, o1)

        def load_idx_blk(blk, ib):
            pltpu.sync_copy(
                idx_hbm.at[pl.ds(base + blk * _IDXBLK, _IDXBLK)],
                idx_v.at[ib])

        def fire_gathers(g, b):
            ib = (g // sup_per_blk) % 2
            loc = (g % sup_per_blk) * _SUP
            for j in range(_SUP // _CHUNK):
                pltpu.async_copy(
                    src_hbm.at[idx_v.at[ib, pl.ds(loc + j * _CHUNK, _CHUNK)]],
                    rows_v.at[b, pl.ds(j * _CHUNK, _CHUNK)],
                    sem_g[b],
                )

        def drain_gathers(b):
            pltpu.make_async_copy(
                src_hbm.at[pl.ds(0, _SUP)], rows_v.at[b], sem_g[b]
            ).wait()

        load_idx_blk(0, 0)
        for b in range(2):
            fire_gathers(b, b)

        def body(i, carry):
            for b in range(2):
                g = 2 * i + b
                drain_gathers(b)
                store = pltpu.async_copy(
                    rows_v.at[b],
                    out_hbm.at[pl.ds(base + g * _SUP, _SUP), pl.ds(0, d)],
                    sem_o[b])

                @pl.when(
                    jnp.logical_and(
                        (g + 2) % sup_per_blk == 0,
                        (g + 2) // sup_per_blk < n_sup // sup_per_blk))
                def _():
                    load_idx_blk((g + 2) // sup_per_blk,
                                 ((g + 2) // sup_per_blk) % 2)

                store.wait()

                @pl.when(g + 2 < n_sup)
                def _():
                    fire_gathers(g + 2, b)

            return carry

        lax.fori_loop(0, n_sup // 2, body, 0)

    return k


def kernel(input_ids, id_map, table):
    b, h = input_ids.shape
    v, d = table.shape
    ids_flat = input_ids.reshape(-1).astype(jnp.int32)
    idmap = id_map.astype(jnp.int32)

    grain = _NW * _CHUNK
    v_pad = ((v + grain - 1) // grain) * grain
    if v_pad != v:
        idmap = jnp.concatenate(
            [idmap, jnp.zeros((v_pad - v,), jnp.int32)])

    permute = _make_permute(v_pad, d)
    ptable = permute(table, idmap)

    gather = _make_gather(b * h, d, _CHUNK)
    out = gather(ptable, ids_flat)
    return out[:, :d].reshape(b, h, d)
